# baseline (speedup 1.0000x reference)
import jax
import jax.numpy as jnp
from jax.experimental import pallas as pl
from jax.experimental.pallas import tpu as pltpu


def _mlp_body(x_ref, a_ref, c_ref, w1_ref, b1_ref, w2_ref, b2_ref, o_ref):
    x = x_ref[...]
    a = a_ref[...]
    h = x[:, 0:1] * a[0:1, :] + x[:, 1:2] * a[1:2, :] + x[:, 2:3] * a[2:3, :]
    h = jnp.maximum(h + c_ref[...], 0.0)
    h2 = jnp.dot(h, w1_ref[...], preferred_element_type=jnp.float32)
    h2 = jnp.maximum(h2 + b1_ref[...], 0.0)
    s = jnp.sum(h2 * w2_ref[...], axis=1, keepdims=True)
    o_ref[...] = s + b2_ref[...]


def kernel(node_feats, edge_index, edge_attr, encoded_vnfs,
           W_node, b_node, W_edge, b_edge,
           W_self0, b_self0, W_neigh0, b_neigh0,
           W_self1, b_self1, W_neigh1, b_neigh1,
           W_att, b_att, W_s1, b_s1, W_s2, b_s2):
    E = edge_attr.shape[0]
    H = W_att.shape[0]

    vnf_mean = jnp.mean(encoded_vnfs, axis=0)
    A = (W_att[:, :H] @ W_edge).T
    c = (W_att[:, :H] @ b_edge + W_att[:, H:] @ vnf_mean + b_att)[None, :]
    W1t = W_s1.T
    w2 = W_s2
    b2 = b_s2[None, :]

    BLK = 6400
    E_pad = pl.cdiv(E, BLK) * BLK
    x = edge_attr
    if E_pad != E:
        x = jnp.pad(x, ((0, E_pad - E), (0, 0)))
    grid = E_pad // BLK

    out = pl.pallas_call(
        _mlp_body,
        grid=(grid,),
        in_specs=[
            pl.BlockSpec((BLK, 3), lambda i: (i, 0)),
            pl.BlockSpec((3, H), lambda i: (0, 0)),
            pl.BlockSpec((1, H), lambda i: (0, 0)),
            pl.BlockSpec((H, H), lambda i: (0, 0)),
            pl.BlockSpec((1, H), lambda i: (0, 0)),
            pl.BlockSpec((1, H), lambda i: (0, 0)),
            pl.BlockSpec((1, 1), lambda i: (0, 0)),
        ],
        out_specs=pl.BlockSpec((BLK, 1), lambda i: (i, 0)),
        out_shape=jax.ShapeDtypeStruct((E_pad, 1), jnp.float32),
        compiler_params=pltpu.CompilerParams(
            dimension_semantics=("parallel",),
        ),
    )(x, A, c, W1t, b_s1[None, :], w2, b2)

    return out[:E, 0]


# speedup vs baseline: 1.7474x; 1.7474x over previous
import jax
import jax.numpy as jnp
from jax.experimental import pallas as pl
from jax.experimental.pallas import tpu as pltpu


def _mlp_body(x_ref, a_ref, c_ref, w1_ref, b1_ref, w2_ref, b2_ref, o_ref):
    x = x_ref[...]
    h = jnp.dot(x, a_ref[...], preferred_element_type=jnp.float32)
    h = jnp.maximum(h + c_ref[...], 0.0)
    h2t = jax.lax.dot_general(w1_ref[...], h, (((1,), (1,)), ((), ())),
                              preferred_element_type=jnp.float32)
    h2t = jnp.maximum(h2t + b1_ref[...], 0.0)
    s = jnp.dot(w2_ref[...], h2t, preferred_element_type=jnp.float32)
    o_ref[...] = s + b2_ref[0]


def kernel(node_feats, edge_index, edge_attr, encoded_vnfs,
           W_node, b_node, W_edge, b_edge,
           W_self0, b_self0, W_neigh0, b_neigh0,
           W_self1, b_self1, W_neigh1, b_neigh1,
           W_att, b_att, W_s1, b_s1, W_s2, b_s2):
    E = edge_attr.shape[0]
    H = W_att.shape[0]

    vnf_mean = jnp.mean(encoded_vnfs, axis=0)
    A = (W_att[:, :H] @ W_edge).T
    c = (W_att[:, :H] @ b_edge + W_att[:, H:] @ vnf_mean + b_att)[None, :]
    w2 = W_s2

    BLK = 6400
    grid = E // BLK

    out = pl.pallas_call(
        _mlp_body,
        grid=(grid,),
        in_specs=[
            pl.BlockSpec((BLK, 3), lambda i: (i, 0)),
            pl.BlockSpec((3, H), lambda i: (0, 0)),
            pl.BlockSpec((1, H), lambda i: (0, 0)),
            pl.BlockSpec((H, H), lambda i: (0, 0)),
            pl.BlockSpec((H, 1), lambda i: (0, 0)),
            pl.BlockSpec((1, H), lambda i: (0, 0)),
            pl.BlockSpec(memory_space=pltpu.SMEM),
        ],
        out_specs=pl.BlockSpec((1, BLK), lambda i: (0, i)),
        out_shape=jax.ShapeDtypeStruct((1, E), jnp.float32),
        compiler_params=pltpu.CompilerParams(
            dimension_semantics=("parallel",),
        ),
    )(edge_attr, A, c, W_s1, b_s1[:, None], w2, b_s2)

    return out[0]


# speedup vs baseline: 7.8052x; 4.4667x over previous
import jax
import jax.numpy as jnp
from jax.experimental import pallas as pl
from jax.experimental.pallas import tpu as pltpu


def _mlp_body(x_ref, a_ref, c_ref, w1_ref, b1_ref, w2_ref, b2_ref, o_ref):
    x_t = x_ref[...]
    h_t = jnp.dot(a_ref[...], x_t, preferred_element_type=jnp.float32)
    h_t = jnp.maximum(h_t + c_ref[...], 0.0)
    h2_t = jnp.dot(w1_ref[...], h_t, preferred_element_type=jnp.float32)
    h2_t = jnp.maximum(h2_t + b1_ref[...], 0.0)
    s = jnp.dot(w2_ref[...], h2_t, preferred_element_type=jnp.float32)
    o_ref[...] = (s + b2_ref[0]).reshape(o_ref.shape)


def kernel(node_feats, edge_index, edge_attr, encoded_vnfs,
           W_node, b_node, W_edge, b_edge,
           W_self0, b_self0, W_neigh0, b_neigh0,
           W_self1, b_self1, W_neigh1, b_neigh1,
           W_att, b_att, W_s1, b_s1, W_s2, b_s2):
    E = edge_attr.shape[0]
    H = W_att.shape[0]

    vnf_mean = jnp.mean(encoded_vnfs, axis=0)
    A2 = W_att[:, :H] @ W_edge
    c = (W_att[:, :H] @ b_edge + W_att[:, H:] @ vnf_mean + b_att)[:, None]

    x_t = edge_attr.T

    BLK = 8192
    grid = pl.cdiv(E, BLK)

    out = pl.pallas_call(
        _mlp_body,
        grid=(grid,),
        in_specs=[
            pl.BlockSpec((3, BLK), lambda i: (0, i)),
            pl.BlockSpec((H, 3), lambda i: (0, 0)),
            pl.BlockSpec((H, 1), lambda i: (0, 0)),
            pl.BlockSpec((H, H), lambda i: (0, 0)),
            pl.BlockSpec((H, 1), lambda i: (0, 0)),
            pl.BlockSpec((1, H), lambda i: (0, 0)),
            pl.BlockSpec(memory_space=pltpu.SMEM),
        ],
        out_specs=pl.BlockSpec((BLK,), lambda i: (i,)),
        out_shape=jax.ShapeDtypeStruct((E,), jnp.float32),
        compiler_params=pltpu.CompilerParams(
            dimension_semantics=("parallel",),
        ),
    )(x_t, A2, c, W_s1, b_s1[:, None], W_s2, b_s2)

    return out
